# baseline (speedup 1.0000x reference)
import functools

import jax
import jax.numpy as jnp
from jax.experimental import pallas as pl
from jax.experimental.pallas import tpu as pltpu

NUM_EXPERTS = 8
TOP_K = 2
HIDDEN = 1024
INTERMEDIATE = 1024
SWIGLU_LIMIT = 7.0
SWIGLU_ALPHA = 1.702
TOKENS = 1024


def _moe_body(rl_ref, x_ref, gup_ref, gub_ref, dp_ref, dpb_ref, out_ref):
    e = pl.program_id(0)
    x = x_ref[...]
    w = gup_ref[0]
    h = jnp.dot(x.astype(jnp.bfloat16), w,
                preferred_element_type=jnp.float32) + gub_ref[0, 0][None, :]
    g = h[:, :INTERMEDIATE]
    l = h[:, INTERMEDIATE:]
    g = jnp.minimum(g, SWIGLU_LIMIT)
    l = jnp.clip(l, -SWIGLU_LIMIT, SWIGLU_LIMIT)
    s = g * jax.nn.sigmoid(SWIGLU_ALPHA * g) * (l + 1.0)
    y = jnp.dot(s.astype(jnp.bfloat16), dp_ref[0],
                preferred_element_type=jnp.float32) + dpb_ref[0, 0][None, :]

    logits = rl_ref[...]
    mx = jnp.max(logits, axis=1, keepdims=True)
    ex = jnp.exp(logits - mx)
    probs = ex / jnp.sum(ex, axis=1, keepdims=True)
    idx = jax.lax.broadcasted_iota(jnp.int32, probs.shape, 1)
    m1 = jnp.max(probs, axis=1, keepdims=True)
    i1 = jnp.min(jnp.where(probs == m1, idx, NUM_EXPERTS), axis=1,
                 keepdims=True)
    p2 = jnp.where(idx == i1, -jnp.inf, probs)
    m2 = jnp.max(p2, axis=1, keepdims=True)
    i2 = jnp.min(jnp.where(p2 == m2, idx, NUM_EXPERTS), axis=1,
                 keepdims=True)
    denom = m1 + m2
    c = jnp.where(i1 == e, m1 / denom, jnp.where(i2 == e, m2 / denom, 0.0))

    @pl.when(e == 0)
    def _():
        out_ref[...] = jnp.zeros_like(out_ref)

    out_ref[...] += c * y


def kernel(hidden_states, router_logits, gate_up_proj, gate_up_proj_bias,
           down_proj, down_proj_bias):
    gup = jnp.concatenate(
        [gate_up_proj[:, :, 0::2], gate_up_proj[:, :, 1::2]], axis=-1
    ).astype(jnp.bfloat16)
    gub = jnp.concatenate(
        [gate_up_proj_bias[:, 0::2], gate_up_proj_bias[:, 1::2]], axis=-1)
    return pl.pallas_call(
        _moe_body,
        grid=(NUM_EXPERTS,),
        in_specs=[
            pl.BlockSpec((TOKENS, NUM_EXPERTS), lambda e: (0, 0)),
            pl.BlockSpec((TOKENS, HIDDEN), lambda e: (0, 0)),
            pl.BlockSpec((1, HIDDEN, 2 * INTERMEDIATE), lambda e: (e, 0, 0)),
            pl.BlockSpec((1, 1, 2 * INTERMEDIATE), lambda e: (e, 0, 0)),
            pl.BlockSpec((1, INTERMEDIATE, HIDDEN), lambda e: (e, 0, 0)),
            pl.BlockSpec((1, 1, HIDDEN), lambda e: (e, 0, 0)),
        ],
        out_specs=pl.BlockSpec((TOKENS, HIDDEN), lambda e: (0, 0)),
        out_shape=jax.ShapeDtypeStruct((TOKENS, HIDDEN), jnp.float32),
    )(router_logits, hidden_states, gup,
      gub[:, None, :], down_proj.astype(jnp.bfloat16),
      down_proj_bias[:, None, :])
